# Initial kernel scaffold; baseline (speedup 1.0000x reference)
import jax
import jax.numpy as jnp
from jax.experimental import pallas as pl


def kernel(x_a, x_b, edge_index_ab, edge_index_ba, W_in_a, b_in_a, W_in_b, b_in_b, W_in2_a, b_in2_a, W_in2_b, b_in2_b, W_g1, att_src1, att_dst1, b_g1, W_g2, att_src2, att_dst2, b_g2, Wq, bq, Wk, bk, Wv, bv, Wo, bo):
    raise NotImplementedError("write your pallas kernel here")


# speedup vs baseline: 29.3572x; 29.3572x over previous
import functools

import jax
import jax.numpy as jnp
from jax import lax
from jax.experimental import pallas as pl
from jax.experimental.pallas import tpu as pltpu
from jax.experimental.pallas import tpu_sc as plsc

F32 = jnp.float32

N_A = 4096
N_B = 4096
DIM = 256
N_NODES = N_A + N_B
PAD = 272
E_DIR = 65536
E_SC = E_DIR + N_A
N_TILES = 16
E_TILE = E_SC // N_TILES
CHUNK = 64
N_CHUNKS = E_TILE // CHUNK
ROWS_PER_TILE = N_A // N_TILES
FBLK = 16

RB = 512


def _prep_all_body(x_ref, w1_ref, b1_ref, w2_ref, b2_ref, wq_ref, bq_ref,
                   wk_ref, bk_ref, wv_ref, bv_ref, l_ref, q_ref, k_ref,
                   v_ref):
    x = x_ref[:]
    l_ref[:] = (jnp.dot(x, w1_ref[0], preferred_element_type=F32)
                + b1_ref[0])
    g = jnp.dot(x, w2_ref[0], preferred_element_type=F32) + b2_ref[0]
    q = jnp.dot(g, wq_ref[:], preferred_element_type=F32) + bq_ref[:]
    q_ref[:] = (q * (1.0 / 16.0)).astype(jnp.bfloat16)
    k = jnp.dot(g, wk_ref[:], preferred_element_type=F32) + bk_ref[:]
    k_ref[:] = k.astype(jnp.bfloat16)
    v = jnp.dot(g, wv_ref[:], preferred_element_type=F32) + bv_ref[:]
    v_ref[:] = v.astype(jnp.bfloat16)


def _prep_all(x_all, w1s, b1s, w2s, b2s, wq_t, bq, wk_t, bk, wv_t, bv):
    grid = (N_NODES // RB,)
    half = (N_NODES // RB) // 2
    blk = pl.BlockSpec((RB, DIM), lambda i: (i, 0))
    wsel = pl.BlockSpec((1, DIM, DIM), lambda i: (i // half, 0, 0))
    bsel = pl.BlockSpec((1, 1, DIM), lambda i: (i // half, 0, 0))
    wspec = pl.BlockSpec((DIM, DIM), lambda i: (0, 0))
    bspec = pl.BlockSpec((1, DIM), lambda i: (0, 0))
    return pl.pallas_call(
        _prep_all_body,
        grid=grid,
        in_specs=[blk, wsel, bsel, wsel, bsel, wspec, bspec, wspec, bspec,
                  wspec, bspec],
        out_specs=[blk, blk, blk, blk],
        out_shape=[jax.ShapeDtypeStruct((N_NODES, DIM), F32)]
        + [jax.ShapeDtypeStruct((N_NODES, DIM), jnp.bfloat16)] * 3,
        interpret=False,
    )(x_all, w1s, b1s, w2s, b2s, wq_t, bq.reshape(1, DIM),
      wk_t, bk.reshape(1, DIM), wv_t, bv.reshape(1, DIM))


KVB = 2048


def _flash_body(q_ref, k_ref, v_ref, wo_ref, bo_ref, o_ref):
    q = q_ref[:]

    def kv_step(kb, carry):
        m_prev, l_prev, acc_prev = carry
        kc = k_ref[pl.ds(kb * KVB, KVB), :]
        s = lax.dot_general(q, kc, (((1,), (1,)), ((), ())),
                            preferred_element_type=F32)
        m_new = jnp.maximum(m_prev, jnp.max(s, axis=1, keepdims=True))
        p = jnp.exp(s - m_new)
        alpha = jnp.exp(m_prev - m_new)
        vc = v_ref[pl.ds(kb * KVB, KVB), :]
        pv = jnp.dot(p.astype(jnp.bfloat16), vc, preferred_element_type=F32)
        l_new = l_prev * alpha + jnp.sum(p, axis=1, keepdims=True)
        acc_new = acc_prev * alpha + pv
        return m_new, l_new, acc_new

    init = (jnp.full((RB, 1), -1e30, F32), jnp.zeros((RB, 1), F32),
            jnp.zeros((RB, DIM), F32))
    m_f, l_f, acc_f = lax.fori_loop(0, N_NODES // KVB, kv_step, init)
    o = acc_f / l_f
    o_ref[:] = jnp.dot(o, wo_ref[:], preferred_element_type=F32) + bo_ref[:]


def _flash(q, k, v, wo_t, bo):
    nq = N_NODES // RB
    return pl.pallas_call(
        _flash_body,
        grid=(nq,),
        in_specs=[
            pl.BlockSpec((RB, DIM), lambda qb: (qb, 0)),
            pl.BlockSpec((N_NODES, DIM), lambda qb: (0, 0)),
            pl.BlockSpec((N_NODES, DIM), lambda qb: (0, 0)),
            pl.BlockSpec((DIM, DIM), lambda qb: (0, 0)),
            pl.BlockSpec((1, DIM), lambda qb: (0, 0)),
        ],
        out_specs=pl.BlockSpec((RB, DIM), lambda qb: (qb, 0)),
        out_shape=jax.ShapeDtypeStruct((N_NODES, DIM), F32),
        interpret=False,
    )(q, k, v, wo_t, bo.reshape(1, DIM))


def _gatpre_body(h_ref, w_ref, att_ref, xwp_ref, adt_ref, m_ref, mx_s):
    i = pl.program_id(0)
    xw = jnp.dot(h_ref[:], w_ref[:], preferred_element_type=F32)
    a = jnp.dot(xw, att_ref[:], preferred_element_type=F32)
    xwp_ref[:, :DIM] = xw
    lane = lax.broadcasted_iota(jnp.int32, (RB, PAD - DIM), 1)
    xwp_ref[:, DIM:] = jnp.where(lane == 0, 1.0,
                                 jnp.where(lane == 1, a[:, 0:1], 0.0))
    adt_ref[:] = jnp.broadcast_to(a[:, 1:2], (RB, 16))

    @pl.when(i == 0)
    def _():
        mx_s[0] = -1e30
        mx_s[1] = -1e30
        mx_s[2] = -1e30

    as_max = jnp.max(a[:, 0])
    ad_max = jnp.max(a[:, 1])
    mx_s[0] = jnp.maximum(mx_s[0], as_max)

    @pl.when(i < (N_NODES // RB) // 2)
    def _():
        mx_s[1] = jnp.maximum(mx_s[1], ad_max)

    @pl.when(i >= (N_NODES // RB) // 2)
    def _():
        mx_s[2] = jnp.maximum(mx_s[2], ad_max)

    @pl.when(i == N_NODES // RB - 1)
    def _():
        t0 = mx_s[0] + mx_s[1]
        t1 = mx_s[0] + mx_s[2]
        m0 = jnp.where(t0 > 0, t0, 0.2 * t0)
        m1 = jnp.where(t1 > 0, t1, 0.2 * t1)
        row = lax.broadcasted_iota(jnp.int32, (8, 16), 0)
        m_ref[:] = jnp.where(row == 0, m0, m1)


def _gatpre(h, w, att_s, att_d):
    grid = (N_NODES // RB,)
    att = jnp.stack([att_s, att_d], axis=1)
    return pl.pallas_call(
        _gatpre_body,
        grid=grid,
        in_specs=[
            pl.BlockSpec((RB, DIM), lambda i: (i, 0)),
            pl.BlockSpec((DIM, DIM), lambda i: (0, 0)),
            pl.BlockSpec((DIM, 2), lambda i: (0, 0)),
        ],
        out_specs=[
            pl.BlockSpec((RB, PAD), lambda i: (i, 0)),
            pl.BlockSpec((RB, 16), lambda i: (i, 0)),
            pl.BlockSpec((8, 16), lambda i: (0, 0)),
        ],
        out_shape=[
            jax.ShapeDtypeStruct((N_NODES, PAD), F32),
            jax.ShapeDtypeStruct((N_NODES, 16), F32),
            jax.ShapeDtypeStruct((8, 16), F32),
        ],
        scratch_shapes=[pltpu.SMEM((4,), F32)],
        interpret=False,
    )(h, w, att)


def _gat_sc_body(xwp_hbm, adt_hbm, m_hbm, src_hbm, dst_hbm, bg_hbm,
                 zeros_hbm, out_hbm, mv, bgv, srcf, dstlf,
                 dstgv0, dstgv1, adr0, adr1, rows0, rows1, fbuf, obuf, acc,
                 semr0, semr1, sema0, sema1):
    c = lax.axis_index("c")
    sid = lax.axis_index("s")

    pltpu.sync_copy(m_hbm.at[c], mv)
    pltpu.sync_copy(bg_hbm, bgv)
    pltpu.sync_copy(zeros_hbm, acc.at[pl.ds(sid * ROWS_PER_TILE,
                                            ROWS_PER_TILE)])

    m = mv[pl.ds(0, 16)]

    plsc.subcore_barrier()

    dstgv = (dstgv0, dstgv1)
    adr = (adr0, adr1)
    rows = (rows0, rows1)
    semr = (semr0, semr1)
    sema = (sema0, sema1)

    pltpu.sync_copy(src_hbm.at[c, sid], srcf)
    pltpu.sync_copy(dst_hbm.at[c, sid], dstlf)

    def prefetch(pb, ci):
        for j in range(CHUNK // 16):
            dstgv[pb][pl.ds(j * 16, 16)] = (dstlf[ci, pl.ds(j * 16, 16)]
                                            + c * N_A)
        pltpu.async_copy(xwp_hbm.at[srcf.at[pl.ds(ci * CHUNK, CHUNK)]],
                         rows[pb], semr[pb])
        pltpu.async_copy(adt_hbm.at[dstgv[pb]], adr[pb], sema[pb])

    def wait_gathers(pb, ci):
        pltpu.make_async_copy(xwp_hbm.at[srcf.at[pl.ds(ci * CHUNK, CHUNK)]],
                              rows[pb], semr[pb]).wait()
        pltpu.make_async_copy(adt_hbm.at[dstgv[pb]], adr[pb], sema[pb]).wait()

    def process(pb, ci):
        @plsc.parallel_loop(0, CHUNK, 1, unroll=8)
        def edge_body(jj):
            av = rows[pb][jj, pl.ds(DIM, 16)]
            dv = adr[pb][jj, pl.ds(0, 16)]
            e = av + dv
            e = jnp.where(e > 0, e, 0.2 * e)
            ex = jnp.exp(e - m)
            sc = ex[1]
            for kk in range(PAD // 16):
                rows[pb][jj, pl.ds(kk * 16, 16)] = (
                    rows[pb][jj, pl.ds(kk * 16, 16)] * sc)

        pltpu.sync_copy(rows[pb], acc.at[dstlf.at[ci]], add=True)

    prefetch(0, 0)

    def pair_body(it, carry):
        prefetch(1, 2 * it + 1)
        wait_gathers(0, 2 * it)
        process(0, 2 * it)

        @pl.when(it < N_CHUNKS // 2 - 1)
        def _():
            prefetch(0, 2 * it + 2)

        wait_gathers(1, 2 * it + 1)
        process(1, 2 * it + 1)
        return carry

    lax.fori_loop(0, N_CHUNKS // 2, pair_body, 0)

    plsc.subcore_barrier()

    def fin_body(fc, carry):
        nb = sid * ROWS_PER_TILE + fc * FBLK
        pltpu.sync_copy(acc.at[pl.ds(nb, FBLK)], fbuf)

        def node_body(jj, carry2):
            r = (1.0 / fbuf[jj, pl.ds(DIM, 16)])[0]
            for kk in range(DIM // 16):
                obuf[jj, pl.ds(kk * 16, 16)] = (
                    fbuf[jj, pl.ds(kk * 16, 16)] * r + bgv[pl.ds(kk * 16, 16)])
            return carry2

        lax.fori_loop(0, FBLK, node_body, 0)
        pltpu.sync_copy(obuf, out_hbm.at[pl.ds(c * N_A + nb, FBLK)])
        return carry

    lax.fori_loop(0, ROWS_PER_TILE // FBLK, fin_body, 0)


@functools.lru_cache(maxsize=1)
def _get_gat_sc():
    return pl.kernel(
        _gat_sc_body,
        out_type=jax.ShapeDtypeStruct((N_NODES, DIM), F32),
        mesh=plsc.VectorSubcoreMesh(core_axis_name="c", subcore_axis_name="s"),
        compiler_params=pltpu.CompilerParams(use_tc_tiling_on_sc=False),
        scratch_types=[
            pltpu.VMEM((16,), F32),
            pltpu.VMEM((DIM,), F32),
            pltpu.VMEM((E_TILE,), jnp.int32),
            pltpu.VMEM((N_CHUNKS, CHUNK), jnp.int32),
            pltpu.VMEM((CHUNK,), jnp.int32),
            pltpu.VMEM((CHUNK,), jnp.int32),
            pltpu.VMEM((CHUNK, 16), F32),
            pltpu.VMEM((CHUNK, 16), F32),
            pltpu.VMEM((CHUNK, PAD), F32),
            pltpu.VMEM((CHUNK, PAD), F32),
            pltpu.VMEM((FBLK, PAD), F32),
            pltpu.VMEM((FBLK, DIM), F32),
            pltpu.VMEM_SHARED((N_A, PAD), F32),
            pltpu.SemaphoreType.DMA,
            pltpu.SemaphoreType.DMA,
            pltpu.SemaphoreType.DMA,
            pltpu.SemaphoreType.DMA,
        ],
    )


def _gat_sc(xwp, adt, m, src_all, dst_all, b_g, zeros):
    return _get_gat_sc()(xwp, adt, m, src_all, dst_all, b_g, zeros)


def _gat_layer(h, w_g, att_s, att_d, b_g, src_all, dst_all, zeros):
    xwp, adt, m = _gatpre(h, w_g, att_s, att_d)
    return _gat_sc(xwp, adt, m, src_all, dst_all, b_g, zeros)


def kernel(x_a, x_b, edge_index_ab, edge_index_ba, W_in_a, b_in_a, W_in_b,
           b_in_b, W_in2_a, b_in2_a, W_in2_b, b_in2_b, W_g1, att_src1,
           att_dst1, b_g1, W_g2, att_src2, att_dst2, b_g2, Wq, bq, Wk, bk,
           Wv, bv, Wo, bo):
    loop_a = jnp.arange(N_A, dtype=jnp.int32)
    loop_b = jnp.arange(N_B, dtype=jnp.int32)
    src0 = jnp.concatenate([edge_index_ba[0] + N_A, loop_a])
    dst0 = jnp.concatenate([edge_index_ba[1], loop_a])
    src1 = jnp.concatenate([edge_index_ab[0], N_A + loop_b])
    dst1 = jnp.concatenate([edge_index_ab[1], loop_b])
    src_all = jnp.stack([src0, src1]).reshape(2, N_TILES, E_TILE)
    dst_all = jnp.stack([dst0, dst1]).reshape(2, N_TILES, N_CHUNKS, CHUNK)
    zeros = jnp.zeros((ROWS_PER_TILE, PAD), F32)

    x_all = jnp.concatenate([x_a, x_b], axis=0)
    w1s = jnp.stack([W_in_a, W_in_b])
    b1s = jnp.stack([b_in_a, b_in_b]).reshape(2, 1, DIM)
    w2s = jnp.stack([W_in2_a, W_in2_b])
    b2s = jnp.stack([b_in2_a, b_in2_b]).reshape(2, 1, DIM)
    l_h, q, k, v = _prep_all(x_all, w1s, b1s, w2s, b2s, Wq.T, bq, Wk.T, bk,
                             Wv.T, bv)

    l_h = _gat_layer(l_h, W_g1, att_src1, att_dst1, b_g1, src_all, dst_all,
                     zeros)
    l_h = _gat_layer(l_h, W_g2, att_src2, att_dst2, b_g2, src_all, dst_all,
                     zeros)

    g_out = _flash(q, k, v, Wo.T, bo)

    z_a = jnp.concatenate([l_h[:N_A], g_out[:N_A]], axis=1)
    z_b = jnp.concatenate([l_h[N_A:], g_out[N_A:]], axis=1)
    return (z_a, z_b)


# speedup vs baseline: 29.7342x; 1.0128x over previous
import functools

import jax
import jax.numpy as jnp
from jax import lax
from jax.experimental import pallas as pl
from jax.experimental.pallas import tpu as pltpu
from jax.experimental.pallas import tpu_sc as plsc

F32 = jnp.float32

N_A = 4096
N_B = 4096
DIM = 256
N_NODES = N_A + N_B
PAD = 272
E_DIR = 65536
E_SC = E_DIR + N_A
N_TILES = 16
E_TILE = E_SC // N_TILES
CHUNK = 64
N_CHUNKS = E_TILE // CHUNK
ROWS_PER_TILE = N_A // N_TILES
FBLK = 16

RB = 512


def _prep_all_body(x_ref, w1_ref, b1_ref, w2_ref, b2_ref, wq_ref, bq_ref,
                   wk_ref, bk_ref, wv_ref, bv_ref, l_ref, q_ref, k_ref,
                   v_ref):
    x = x_ref[:]
    l_ref[:] = (jnp.dot(x, w1_ref[0], preferred_element_type=F32)
                + b1_ref[0])
    g = jnp.dot(x, w2_ref[0], preferred_element_type=F32) + b2_ref[0]
    q = jnp.dot(g, wq_ref[:], preferred_element_type=F32) + bq_ref[:]
    q_ref[:] = (q * (1.0 / 16.0)).astype(jnp.bfloat16)
    k = jnp.dot(g, wk_ref[:], preferred_element_type=F32) + bk_ref[:]
    k_ref[:] = k.astype(jnp.bfloat16)
    v = jnp.dot(g, wv_ref[:], preferred_element_type=F32) + bv_ref[:]
    v_ref[:] = v.astype(jnp.bfloat16)


def _prep_all(x_all, w1s, b1s, w2s, b2s, wq_t, bq, wk_t, bk, wv_t, bv):
    grid = (N_NODES // RB,)
    half = (N_NODES // RB) // 2
    blk = pl.BlockSpec((RB, DIM), lambda i: (i, 0))
    wsel = pl.BlockSpec((1, DIM, DIM), lambda i: (i // half, 0, 0))
    bsel = pl.BlockSpec((1, 1, DIM), lambda i: (i // half, 0, 0))
    wspec = pl.BlockSpec((DIM, DIM), lambda i: (0, 0))
    bspec = pl.BlockSpec((1, DIM), lambda i: (0, 0))
    return pl.pallas_call(
        _prep_all_body,
        grid=grid,
        in_specs=[blk, wsel, bsel, wsel, bsel, wspec, bspec, wspec, bspec,
                  wspec, bspec],
        out_specs=[blk, blk, blk, blk],
        out_shape=[jax.ShapeDtypeStruct((N_NODES, DIM), F32)]
        + [jax.ShapeDtypeStruct((N_NODES, DIM), jnp.bfloat16)] * 3,
        interpret=False,
    )(x_all, w1s, b1s, w2s, b2s, wq_t, bq.reshape(1, DIM),
      wk_t, bk.reshape(1, DIM), wv_t, bv.reshape(1, DIM))


KVB = 4096


def _flash_body(q_ref, k_ref, v_ref, wo_ref, bo_ref, o_ref):
    q = q_ref[:]

    def kv_step(kb, carry):
        m_prev, l_prev, acc_prev = carry
        kc = k_ref[pl.ds(kb * KVB, KVB), :]
        s = lax.dot_general(q, kc, (((1,), (1,)), ((), ())),
                            preferred_element_type=F32)
        m_new = jnp.maximum(m_prev, jnp.max(s, axis=1, keepdims=True))
        p = jnp.exp(s - m_new)
        alpha = jnp.exp(m_prev - m_new)
        vc = v_ref[pl.ds(kb * KVB, KVB), :]
        pv = jnp.dot(p.astype(jnp.bfloat16), vc, preferred_element_type=F32)
        l_new = l_prev * alpha + jnp.sum(p, axis=1, keepdims=True)
        acc_new = acc_prev * alpha + pv
        return m_new, l_new, acc_new

    init = (jnp.full((RB, 1), -1e30, F32), jnp.zeros((RB, 1), F32),
            jnp.zeros((RB, DIM), F32))
    m_f, l_f, acc_f = lax.fori_loop(0, N_NODES // KVB, kv_step, init)
    o = acc_f / l_f
    o_ref[:] = jnp.dot(o, wo_ref[:], preferred_element_type=F32) + bo_ref[:]


def _flash(q, k, v, wo_t, bo):
    nq = N_NODES // RB
    return pl.pallas_call(
        _flash_body,
        grid=(nq,),
        in_specs=[
            pl.BlockSpec((RB, DIM), lambda qb: (qb, 0)),
            pl.BlockSpec((N_NODES, DIM), lambda qb: (0, 0)),
            pl.BlockSpec((N_NODES, DIM), lambda qb: (0, 0)),
            pl.BlockSpec((DIM, DIM), lambda qb: (0, 0)),
            pl.BlockSpec((1, DIM), lambda qb: (0, 0)),
        ],
        out_specs=pl.BlockSpec((RB, DIM), lambda qb: (qb, 0)),
        out_shape=jax.ShapeDtypeStruct((N_NODES, DIM), F32),
        interpret=False,
    )(q, k, v, wo_t, bo.reshape(1, DIM))


def _gatpre_body(h_ref, w_ref, att_ref, xwp_ref, adt_ref, m_ref, mx_s):
    i = pl.program_id(0)
    xw = jnp.dot(h_ref[:], w_ref[:], preferred_element_type=F32)
    a = jnp.dot(xw, att_ref[:], preferred_element_type=F32)
    xwp_ref[:, :DIM] = xw
    lane = lax.broadcasted_iota(jnp.int32, (RB, PAD - DIM), 1)
    xwp_ref[:, DIM:] = jnp.where(lane == 0, 1.0,
                                 jnp.where(lane == 1, a[:, 0:1], 0.0))
    adt_ref[:] = jnp.broadcast_to(a[:, 1:2], (RB, 16))

    @pl.when(i == 0)
    def _():
        mx_s[0] = -1e30
        mx_s[1] = -1e30
        mx_s[2] = -1e30

    as_max = jnp.max(a[:, 0])
    ad_max = jnp.max(a[:, 1])
    mx_s[0] = jnp.maximum(mx_s[0], as_max)

    @pl.when(i < (N_NODES // RB) // 2)
    def _():
        mx_s[1] = jnp.maximum(mx_s[1], ad_max)

    @pl.when(i >= (N_NODES // RB) // 2)
    def _():
        mx_s[2] = jnp.maximum(mx_s[2], ad_max)

    @pl.when(i == N_NODES // RB - 1)
    def _():
        t0 = mx_s[0] + mx_s[1]
        t1 = mx_s[0] + mx_s[2]
        m0 = jnp.where(t0 > 0, t0, 0.2 * t0)
        m1 = jnp.where(t1 > 0, t1, 0.2 * t1)
        row = lax.broadcasted_iota(jnp.int32, (8, 16), 0)
        m_ref[:] = jnp.where(row == 0, m0, m1)


def _gatpre(h, w, att_s, att_d):
    grid = (N_NODES // RB,)
    att = jnp.stack([att_s, att_d], axis=1)
    return pl.pallas_call(
        _gatpre_body,
        grid=grid,
        in_specs=[
            pl.BlockSpec((RB, DIM), lambda i: (i, 0)),
            pl.BlockSpec((DIM, DIM), lambda i: (0, 0)),
            pl.BlockSpec((DIM, 2), lambda i: (0, 0)),
        ],
        out_specs=[
            pl.BlockSpec((RB, PAD), lambda i: (i, 0)),
            pl.BlockSpec((RB, 16), lambda i: (i, 0)),
            pl.BlockSpec((8, 16), lambda i: (0, 0)),
        ],
        out_shape=[
            jax.ShapeDtypeStruct((N_NODES, PAD), F32),
            jax.ShapeDtypeStruct((N_NODES, 16), F32),
            jax.ShapeDtypeStruct((8, 16), F32),
        ],
        scratch_shapes=[pltpu.SMEM((4,), F32)],
        interpret=False,
    )(h, w, att)


def _gat_sc_body(xwp_hbm, adt_hbm, m_hbm, src_hbm, dst_hbm, bg_hbm,
                 zeros_hbm, out_hbm, mv, bgv, srcf, dstlf,
                 dstgv0, dstgv1, adr0, adr1, rows0, rows1, fbuf, obuf, acc,
                 semr0, semr1, sema0, sema1):
    c = lax.axis_index("c")
    sid = lax.axis_index("s")

    pltpu.sync_copy(m_hbm.at[c], mv)
    pltpu.sync_copy(bg_hbm, bgv)
    pltpu.sync_copy(zeros_hbm, acc.at[pl.ds(sid * ROWS_PER_TILE,
                                            ROWS_PER_TILE)])

    m = mv[pl.ds(0, 16)]

    plsc.subcore_barrier()

    dstgv = (dstgv0, dstgv1)
    adr = (adr0, adr1)
    rows = (rows0, rows1)
    semr = (semr0, semr1)
    sema = (sema0, sema1)

    pltpu.sync_copy(src_hbm.at[c, sid], srcf)
    pltpu.sync_copy(dst_hbm.at[c, sid], dstlf)

    def prefetch(pb, ci):
        for j in range(CHUNK // 16):
            dstgv[pb][pl.ds(j * 16, 16)] = (dstlf[ci, pl.ds(j * 16, 16)]
                                            + c * N_A)
        pltpu.async_copy(xwp_hbm.at[srcf.at[pl.ds(ci * CHUNK, CHUNK)]],
                         rows[pb], semr[pb])
        pltpu.async_copy(adt_hbm.at[dstgv[pb]], adr[pb], sema[pb])

    def wait_gathers(pb, ci):
        pltpu.make_async_copy(xwp_hbm.at[srcf.at[pl.ds(ci * CHUNK, CHUNK)]],
                              rows[pb], semr[pb]).wait()
        pltpu.make_async_copy(adt_hbm.at[dstgv[pb]], adr[pb], sema[pb]).wait()

    def process(pb, ci):
        @plsc.parallel_loop(0, CHUNK, 1, unroll=8)
        def edge_body(jj):
            av = rows[pb][jj, pl.ds(DIM, 16)]
            dv = adr[pb][jj, pl.ds(0, 16)]
            e = av + dv
            e = jnp.where(e > 0, e, 0.2 * e)
            ex = jnp.exp(e - m)
            sc = ex[1]
            for kk in range(PAD // 16):
                rows[pb][jj, pl.ds(kk * 16, 16)] = (
                    rows[pb][jj, pl.ds(kk * 16, 16)] * sc)

        pltpu.sync_copy(rows[pb], acc.at[dstlf.at[ci]], add=True)

    prefetch(0, 0)

    def pair_body(it, carry):
        prefetch(1, 2 * it + 1)
        wait_gathers(0, 2 * it)
        process(0, 2 * it)

        @pl.when(it < N_CHUNKS // 2 - 1)
        def _():
            prefetch(0, 2 * it + 2)

        wait_gathers(1, 2 * it + 1)
        process(1, 2 * it + 1)
        return carry

    lax.fori_loop(0, N_CHUNKS // 2, pair_body, 0)

    plsc.subcore_barrier()

    def fin_body(fc, carry):
        nb = sid * ROWS_PER_TILE + fc * FBLK
        pltpu.sync_copy(acc.at[pl.ds(nb, FBLK)], fbuf)

        def node_body(jj, carry2):
            r = (1.0 / fbuf[jj, pl.ds(DIM, 16)])[0]
            for kk in range(DIM // 16):
                obuf[jj, pl.ds(kk * 16, 16)] = (
                    fbuf[jj, pl.ds(kk * 16, 16)] * r + bgv[pl.ds(kk * 16, 16)])
            return carry2

        lax.fori_loop(0, FBLK, node_body, 0)
        pltpu.sync_copy(obuf, out_hbm.at[pl.ds(c * N_A + nb, FBLK)])
        return carry

    lax.fori_loop(0, ROWS_PER_TILE // FBLK, fin_body, 0)


@functools.lru_cache(maxsize=1)
def _get_gat_sc():
    return pl.kernel(
        _gat_sc_body,
        out_type=jax.ShapeDtypeStruct((N_NODES, DIM), F32),
        mesh=plsc.VectorSubcoreMesh(core_axis_name="c", subcore_axis_name="s"),
        compiler_params=pltpu.CompilerParams(use_tc_tiling_on_sc=False),
        scratch_types=[
            pltpu.VMEM((16,), F32),
            pltpu.VMEM((DIM,), F32),
            pltpu.VMEM((E_TILE,), jnp.int32),
            pltpu.VMEM((N_CHUNKS, CHUNK), jnp.int32),
            pltpu.VMEM((CHUNK,), jnp.int32),
            pltpu.VMEM((CHUNK,), jnp.int32),
            pltpu.VMEM((CHUNK, 16), F32),
            pltpu.VMEM((CHUNK, 16), F32),
            pltpu.VMEM((CHUNK, PAD), F32),
            pltpu.VMEM((CHUNK, PAD), F32),
            pltpu.VMEM((FBLK, PAD), F32),
            pltpu.VMEM((FBLK, DIM), F32),
            pltpu.VMEM_SHARED((N_A, PAD), F32),
            pltpu.SemaphoreType.DMA,
            pltpu.SemaphoreType.DMA,
            pltpu.SemaphoreType.DMA,
            pltpu.SemaphoreType.DMA,
        ],
    )


def _gat_sc(xwp, adt, m, src_all, dst_all, b_g, zeros):
    return _get_gat_sc()(xwp, adt, m, src_all, dst_all, b_g, zeros)


def _gat_layer(h, w_g, att_s, att_d, b_g, src_all, dst_all, zeros):
    xwp, adt, m = _gatpre(h, w_g, att_s, att_d)
    return _gat_sc(xwp, adt, m, src_all, dst_all, b_g, zeros)


def kernel(x_a, x_b, edge_index_ab, edge_index_ba, W_in_a, b_in_a, W_in_b,
           b_in_b, W_in2_a, b_in2_a, W_in2_b, b_in2_b, W_g1, att_src1,
           att_dst1, b_g1, W_g2, att_src2, att_dst2, b_g2, Wq, bq, Wk, bk,
           Wv, bv, Wo, bo):
    loop_a = jnp.arange(N_A, dtype=jnp.int32)
    loop_b = jnp.arange(N_B, dtype=jnp.int32)
    src0 = jnp.concatenate([edge_index_ba[0] + N_A, loop_a])
    dst0 = jnp.concatenate([edge_index_ba[1], loop_a])
    src1 = jnp.concatenate([edge_index_ab[0], N_A + loop_b])
    dst1 = jnp.concatenate([edge_index_ab[1], loop_b])
    src_all = jnp.stack([src0, src1]).reshape(2, N_TILES, E_TILE)
    dst_all = jnp.stack([dst0, dst1]).reshape(2, N_TILES, N_CHUNKS, CHUNK)
    zeros = jnp.zeros((ROWS_PER_TILE, PAD), F32)

    x_all = jnp.concatenate([x_a, x_b], axis=0)
    w1s = jnp.stack([W_in_a, W_in_b])
    b1s = jnp.stack([b_in_a, b_in_b]).reshape(2, 1, DIM)
    w2s = jnp.stack([W_in2_a, W_in2_b])
    b2s = jnp.stack([b_in2_a, b_in2_b]).reshape(2, 1, DIM)
    l_h, q, k, v = _prep_all(x_all, w1s, b1s, w2s, b2s, Wq.T, bq, Wk.T, bk,
                             Wv.T, bv)

    l_h = _gat_layer(l_h, W_g1, att_src1, att_dst1, b_g1, src_all, dst_all,
                     zeros)
    l_h = _gat_layer(l_h, W_g2, att_src2, att_dst2, b_g2, src_all, dst_all,
                     zeros)

    g_out = _flash(q, k, v, Wo.T, bo)

    z_a = jnp.concatenate([l_h[:N_A], g_out[:N_A]], axis=1)
    z_b = jnp.concatenate([l_h[N_A:], g_out[N_A:]], axis=1)
    return (z_a, z_b)
